# baseline (speedup 1.0000x reference)
import functools

import jax
import jax.numpy as jnp
from jax import lax
from jax.experimental import pallas as pl
from jax.experimental.pallas import tpu as pltpu
from jax.experimental.pallas import tpu_sc as plsc

D_MODEL = 64
SCALE = 8.0
NUM_CORES = 2
NUM_SUBCORES = 16
NUM_WORKERS = NUM_CORES * NUM_SUBCORES
BATCH = 16384
B_PER_W = BATCH // NUM_WORKERS
LANES = 16
VOCAB = 1_000_000
GRANS_PER_D = VOCAB // LANES
IDX_CHUNK = 32
N_CHUNKS = B_PER_W // IDX_CHUNK
ENTRIES_PER_CHUNK = IDX_CHUNK * D_MODEL
ESTREAMS = ENTRIES_PER_CHUNK // 128


def _build():
    mesh = plsc.VectorSubcoreMesh(core_axis_name="c", subcore_axis_name="s")

    @functools.partial(
        pl.kernel,
        mesh=mesh,
        out_type=jax.ShapeDtypeStruct((BATCH, D_MODEL), jnp.float32),
        scratch_types=[
            pltpu.VMEM((B_PER_W,), jnp.int32),
            pltpu.VMEM((B_PER_W,), jnp.int32),
            pltpu.VMEM((ENTRIES_PER_CHUNK,), jnp.int32),
            pltpu.VMEM((ENTRIES_PER_CHUNK, LANES), jnp.float32),
            pltpu.VMEM((B_PER_W, D_MODEL), jnp.float32),
            pltpu.SemaphoreType.DMA,
        ],
        compiler_params=pltpu.CompilerParams(
            use_tc_tiling_on_sc=False, needs_layout_passes=False
        ),
    )
    def gather_gran(tabT_hbm, idx_hbm, out_hbm, idx_v, gbase_v, elist_v,
                    grans_v, rows_v, sem):
        wid = lax.axis_index("s") * NUM_CORES + lax.axis_index("c")
        b0 = wid * B_PER_W
        pltpu.sync_copy(idx_hbm.at[pl.ds(b0, B_PER_W)], idx_v)

        gtab = tabT_hbm

        for a in range(B_PER_W // LANES):
            v = idx_v[pl.ds(a * LANES, LANES)]
            gbase_v[pl.ds(a * LANES, LANES)] = jnp.right_shift(v, 4)

        offs = []
        for q in range(D_MODEL // LANES):
            dd = lax.iota(jnp.int32, LANES) + (q * LANES)
            offs.append(dd * GRANS_PER_D)

        lane_ids = [jnp.full((LANES,), l, jnp.int32) for l in range(LANES)]
        iota16 = lax.iota(jnp.int32, LANES)

        def chunk_body(ch, carry):
            i0 = ch * IDX_CHUNK
            for al in range(IDX_CHUNK // LANES):
                bvec = gbase_v[pl.ds(i0 + al * LANES, LANES)]
                for l in range(LANES):
                    b_s = jnp.take(bvec, lane_ids[l])
                    e0 = (al * LANES + l) * D_MODEL
                    for q in range(D_MODEL // LANES):
                        elist_v[pl.ds(e0 + q * LANES, LANES)] = b_s + offs[q]
            copies = []
            for j in range(ESTREAMS):
                sl = pl.ds(j * 128, 128)
                copies.append(
                    pltpu.async_copy(
                        gtab.at[elist_v.at[sl]],
                        grans_v.at[pl.ds(j * 128, 128), :],
                        sem,
                    )
                )
            for cp in copies:
                cp.wait()
            for al in range(IDX_CHUNK // LANES):
                vvec = idx_v[pl.ds(i0 + al * LANES, LANES)]
                lvec = jnp.bitwise_and(vvec, 15)
                for l in range(LANES):
                    lane_s = jnp.take(lvec, lane_ids[l])
                    r0 = (al * LANES + l) * D_MODEL
                    for q in range(D_MODEL // LANES):
                        rows = iota16 + (r0 + q * LANES)
                        vals = plsc.load_gather(grans_v, [rows, lane_s])
                        rows_v[i0 + al * LANES + l,
                               pl.ds(q * LANES, LANES)] = vals * SCALE
            return carry

        lax.fori_loop(0, N_CHUNKS, chunk_body, 0)

        pltpu.sync_copy(rows_v, out_hbm.at[pl.ds(b0, B_PER_W), :])

    return gather_gran


_gather_gran = _build()


@jax.jit
def kernel(spec, table):
    idx = spec.reshape(-1).astype(jnp.int32)
    return _gather_gran(table.T.reshape(D_MODEL * GRANS_PER_D, LANES), idx)


# speedup vs baseline: 1.0056x; 1.0056x over previous
import functools

import jax
import jax.numpy as jnp
from jax import lax
from jax.experimental import pallas as pl
from jax.experimental.pallas import tpu as pltpu
from jax.experimental.pallas import tpu_sc as plsc

D_MODEL = 64
SCALE = 8.0
NUM_CORES = 2
NUM_SUBCORES = 16
NUM_WORKERS = NUM_CORES * NUM_SUBCORES
BATCH = 16384
B_PER_W = BATCH // NUM_WORKERS
LANES = 16
VOCAB = 1_000_000
N_ENTRIES = B_PER_W * D_MODEL
ESTREAMS = N_ENTRIES // 128


def _build():
    mesh = plsc.VectorSubcoreMesh(core_axis_name="c", subcore_axis_name="s")

    @functools.partial(
        pl.kernel,
        mesh=mesh,
        out_type=jax.ShapeDtypeStruct((BATCH * D_MODEL // 128, 128), jnp.float32),
        scratch_types=[
            pltpu.VMEM((B_PER_W,), jnp.int32),
            pltpu.VMEM((N_ENTRIES,), jnp.int32),
            pltpu.VMEM((N_ENTRIES // 128, 128), jnp.float32),
            pltpu.SemaphoreType.DMA,
        ],
        compiler_params=pltpu.CompilerParams(
            use_tc_tiling_on_sc=False, needs_layout_passes=False
        ),
    )
    def gather_elem(tflat_hbm, idx_hbm, out_hbm, idx_v, elist_v, rows_v, sem):
        wid = lax.axis_index("s") * NUM_CORES + lax.axis_index("c")
        b0 = wid * B_PER_W
        pltpu.sync_copy(idx_hbm.at[pl.ds(b0, B_PER_W)], idx_v)

        offs = []
        for q in range(D_MODEL // LANES):
            dd = lax.iota(jnp.int32, LANES) + (q * LANES)
            offs.append(dd * VOCAB)

        lane_ids = [jnp.full((LANES,), l, jnp.int32) for l in range(LANES)]

        def build_body(a, carry):
            vvec = idx_v[pl.ds(a * LANES, LANES)]
            for l in range(LANES):
                v_s = jnp.take(vvec, lane_ids[l])
                e0 = (a * LANES + l) * D_MODEL
                for q in range(D_MODEL // LANES):
                    elist_v[pl.ds(e0 + q * LANES, LANES)] = v_s + offs[q]
            return carry

        lax.fori_loop(0, B_PER_W // LANES, build_body, 0)

        copies = []
        for j in range(ESTREAMS):
            sl = pl.ds(j * 128, 128)
            copies.append(
                pltpu.async_copy(
                    tflat_hbm.at[elist_v.at[sl]],
                    rows_v.at[j, :],
                    sem,
                )
            )
        for cp in copies:
            cp.wait()

        def scale_body(r, carry):
            for q in range(128 // LANES):
                sl = pl.ds(q * LANES, LANES)
                rows_v[r, sl] = rows_v[r, sl] * SCALE
            return carry

        lax.fori_loop(0, N_ENTRIES // 128, scale_body, 0)

        pltpu.sync_copy(
            rows_v, out_hbm.at[pl.ds(wid * (N_ENTRIES // 128), N_ENTRIES // 128), :]
        )

    return gather_elem


_gather_elem = _build()


@jax.jit
def kernel(spec, table):
    idx = spec.reshape(-1).astype(jnp.int32)
    tflat = table.T.reshape(D_MODEL * VOCAB)
    out2d = _gather_elem(tflat, idx)
    return out2d.reshape(BATCH, D_MODEL)


# speedup vs baseline: 30.4896x; 30.3185x over previous
import functools

import jax
import jax.numpy as jnp
from jax import lax
from jax.experimental import pallas as pl
from jax.experimental.pallas import tpu as pltpu
from jax.experimental.pallas import tpu_sc as plsc

D_MODEL = 64
SCALE = 8.0
NUM_CORES = 2
NUM_SUBCORES = 16
NUM_WORKERS = NUM_CORES * NUM_SUBCORES
BATCH = 16384
LANES = 16
VOCAB = 1_000_000
VT = (VOCAB + 127) // 128
BASE_T = VT // NUM_WORKERS
EXTRA = VT - BASE_T * NUM_WORKERS
ST = 4
SEC_W = ST * 128
NSEC = (BASE_T + 1 + ST - 1) // ST
FLUSH = 64


def _build():
    mesh = plsc.VectorSubcoreMesh(core_axis_name="c", subcore_axis_name="s")

    @functools.partial(
        pl.kernel,
        mesh=mesh,
        out_type=jax.ShapeDtypeStruct((BATCH, D_MODEL), jnp.float32),
        scratch_types=[
            pltpu.VMEM((BATCH,), jnp.int32),
            pltpu.VMEM((BATCH + LANES,), jnp.int32),
            pltpu.VMEM((BATCH + LANES,), jnp.int32),
            pltpu.VMEM((D_MODEL, SEC_W), jnp.float32),
            pltpu.VMEM((D_MODEL, SEC_W), jnp.float32),
            pltpu.VMEM((FLUSH, D_MODEL), jnp.float32),
            pltpu.SemaphoreType.DMA,
            pltpu.SemaphoreType.DMA,
            pltpu.SemaphoreType.DMA,
        ],
        compiler_params=pltpu.CompilerParams(
            use_tc_tiling_on_sc=True, needs_layout_passes=False
        ),
    )
    def sweep(tabT_hbm, idx_hbm, out_hbm, idx_v, hv_v, sh_v, sec_a, sec_b,
              ostage_v, sem_a, sem_b, sem_o):
        wid = lax.axis_index("s") * NUM_CORES + lax.axis_index("c")
        t_lo = BASE_T * wid + jnp.minimum(wid, EXTRA)
        nt = BASE_T + (wid < EXTRA).astype(jnp.int32)
        t_hi = t_lo + nt
        col_lim = nt * 128

        pltpu.sync_copy(idx_hbm, idx_v)

        iota16 = lax.iota(jnp.int32, LANES)

        def scan_body(a, hcnt):
            v = idx_v[pl.ds(a * LANES, LANES)]
            t = jnp.right_shift(v, 7)
            m = jnp.logical_and(t >= t_lo, t < t_hi)
            packed = jnp.bitwise_or(
                jnp.left_shift(v - t_lo * 128, 14), a * LANES + iota16
            )
            plsc.store_compressed(hv_v.at[pl.ds(hcnt, LANES)], packed, mask=m)
            return hcnt + jnp.sum(m.astype(jnp.int32))

        hcnt = lax.fori_loop(0, BATCH // LANES, scan_body, 0)
        n_hvec = lax.div(hcnt + LANES - 1, LANES)

        def fire_section(s, bufs, sems):
            st = jnp.minimum(t_lo + s * ST, t_hi - ST)
            cps = []
            for bb in range(8):
                cps.append(
                    pltpu.async_copy(
                        tabT_hbm.at[pl.ds(bb * 8, 8), pl.ds(st * 128, SEC_W)],
                        bufs.at[pl.ds(bb * 8, 8), :],
                        sems,
                    )
                )
            return st, cps

        rows_q = [iota16 + q * LANES for q in range(D_MODEL // LANES)]

        def process_section(s, buf):
            st = jnp.minimum(t_lo + s * ST, t_hi - ST)
            st_col = (st - t_lo) * 128
            nom_lo = s * SEC_W
            nom_hi = jnp.minimum(nom_lo + SEC_W, col_lim)

            def rescan_body(j, scnt):
                hp = hv_v[pl.ds(j * LANES, LANES)]
                cr = jnp.right_shift(hp, 14)
                m = jnp.logical_and(cr >= nom_lo, cr < nom_hi)
                m = jnp.logical_and(m, j * LANES + iota16 < hcnt)
                plsc.store_compressed(sh_v.at[pl.ds(scnt, LANES)], hp, mask=m)
                return scnt + jnp.sum(m.astype(jnp.int32))

            scnt = lax.fori_loop(0, n_hvec, rescan_body, 0)

            def batch_body(g, carry):
                cnt = jnp.minimum(scnt - g * FLUSH, FLUSH)

                def ext_body(k, c2):
                    hp = sh_v[pl.ds(g * FLUSH + k, LANES)][0]
                    col = jnp.right_shift(hp, 14) - st_col
                    cols = jnp.full((LANES,), col, jnp.int32)
                    for q in range(D_MODEL // LANES):
                        vals = plsc.load_gather(buf, [rows_q[q], cols])
                        ostage_v[k, pl.ds(q * LANES, LANES)] = vals * SCALE
                    return c2

                lax.fori_loop(0, cnt, ext_body, 0)

                def fire_body(r, c2):
                    hp = sh_v[pl.ds(g * FLUSH + r, LANES)][0]
                    b = jnp.bitwise_and(hp, 16383)
                    pltpu.async_copy(
                        ostage_v.at[pl.ds(r, 1), :],
                        out_hbm.at[pl.ds(b, 1), :],
                        sem_o,
                    )
                    return c2

                lax.fori_loop(0, cnt, fire_body, 0)

                def drain_body(r, c2):
                    pltpu.make_async_copy(
                        ostage_v.at[pl.ds(0, 1), :],
                        out_hbm.at[pl.ds(0, 1), :],
                        sem_o,
                    ).wait()
                    return c2

                lax.fori_loop(0, cnt, drain_body, 0)
                return carry

            lax.fori_loop(0, lax.div(scnt + FLUSH - 1, FLUSH), batch_body, 0)

        def drain_sec(sems):
            for bb in range(8):
                pltpu.make_async_copy(
                    tabT_hbm.at[pl.ds(bb * 8, 8), pl.ds(0, SEC_W)],
                    sec_a.at[pl.ds(bb * 8, 8), :],
                    sems,
                ).wait()

        fire_section(0, sec_a, sem_a)

        def pair_body(p, carry):
            drain_sec(sem_a)
            fire_section(2 * p + 1, sec_b, sem_b)
            process_section(2 * p, sec_a)
            drain_sec(sem_b)
            fire_section(jnp.minimum(2 * p + 2, NSEC - 1), sec_a, sem_a)
            process_section(2 * p + 1, sec_b)
            return carry

        lax.fori_loop(0, NSEC // 2, pair_body, 0)
        drain_sec(sem_a)

    return sweep


_sweep = _build()


@jax.jit
def kernel(spec, table):
    idx = spec.reshape(-1).astype(jnp.int32)
    return _sweep(table.T, idx)
